# baseline (speedup 1.0000x reference)
import functools

import jax
import jax.numpy as jnp
from jax import lax
from jax.experimental import pallas as pl
from jax.experimental.pallas import tpu as pltpu
from jax.experimental.pallas import tpu_sc as plsc

B = 64
L = 8192
HIST = 50
HP = 56
FUSE_WEIGHT = 0.5

ROWS = 2 * B
RPW = 8
NWORK = ROWS // RPW

NK = 16
KB = L // NK


def _idx_body(t2t_ref, tl_ref, out_ref):
    t2t = t2t_ref[...]
    tl = tl_ref[...]
    hio = lax.broadcasted_iota(jnp.int32, (HP, B), 0)
    pos1 = tl - 1
    pos2 = tl - 2 + jnp.where(tl < 2, HIST, 0)
    i1 = jnp.sum(jnp.where(hio == pos1, t2t, 0), axis=0, keepdims=True) - 1
    i2 = jnp.sum(jnp.where(hio == pos2, t2t, 0), axis=0, keepdims=True) - 1
    zero = jnp.zeros((6, B), jnp.int32)
    out_ref[...] = jnp.concatenate([i1, i2, zero], axis=0)


def _compute_idx(traj2t, tl_row):
    out = pl.pallas_call(
        _idx_body,
        out_shape=jax.ShapeDtypeStruct((8, B), jnp.int32),
    )(traj2t, tl_row)
    return out[:2].reshape(ROWS)


def _sc_gather(attMap_e, idx_flat):
    mesh = plsc.VectorSubcoreMesh(core_axis_name="c", subcore_axis_name="s")

    @functools.partial(
        pl.kernel,
        out_type=jax.ShapeDtypeStruct((ROWS, L), jnp.float32),
        mesh=mesh,
        scratch_types=[
            pltpu.VMEM((RPW,), jnp.int32),
            pltpu.VMEM((RPW, L), jnp.float32),
            pltpu.SemaphoreType.DMA,
        ],
    )
    def gather_kernel(att_hbm, idx_hbm, out_hbm, idx_v, rows_v, sem):
        wid = lax.axis_index("s") * 2 + lax.axis_index("c")

        @pl.when(wid < NWORK)
        def _():
            base = wid * RPW
            pltpu.sync_copy(idx_hbm.at[pl.ds(base, RPW)], idx_v)
            pltpu.async_copy(att_hbm.at[idx_v], rows_v, sem).wait()
            pltpu.sync_copy(rows_v, out_hbm.at[pl.ds(base, RPW)])

    return gather_kernel(attMap_e, idx_flat)


def _tc_body(tl_ref, prob1_ref, probk_ref, att_ref, adj_ref, out_ref):
    k = pl.program_id(0)

    @pl.when(k == 0)
    def _():
        out_ref[...] = jnp.zeros_like(out_ref)

    a = probk_ref[...].astype(jnp.bfloat16)
    bm = att_ref[...].astype(jnp.bfloat16)
    out_ref[...] += jnp.dot(a, bm, preferred_element_type=jnp.float32)

    @pl.when(k == NK - 1)
    def _():
        acc = out_ref[...]
        mn2 = jnp.min(acc, axis=-1, keepdims=True)
        mx2 = jnp.max(acc, axis=-1, keepdims=True)
        p2 = (acc - mn2) / (mx2 - mn2)
        p1 = prob1_ref[...]
        mn1 = jnp.min(p1, axis=-1, keepdims=True)
        mx1 = jnp.max(p1, axis=-1, keepdims=True)
        y1 = (p1 - mn1) / (mx1 - mn1)
        adj = adj_ref[...]
        wn = (adj - jnp.min(adj)) / (jnp.max(adj) - jnp.min(adj))
        mask = (tl_ref[...] >= 2).astype(jnp.float32)
        out_ref[...] = y1 + mask * (FUSE_WEIGHT * wn) * p2


def _tc_matmul(gathered, attMap_e, adjust2, tl2d):
    return pl.pallas_call(
        _tc_body,
        grid=(NK,),
        in_specs=[
            pl.BlockSpec((B, 1), lambda k: (0, 0)),
            pl.BlockSpec((B, L), lambda k: (0, 0)),
            pl.BlockSpec((B, KB), lambda k: (1, k)),
            pl.BlockSpec((KB, L), lambda k: (k, 0)),
            pl.BlockSpec((1, L), lambda k: (0, 0)),
        ],
        out_specs=pl.BlockSpec((B, L), lambda k: (0, 0)),
        out_shape=jax.ShapeDtypeStruct((B, L), jnp.float32),
    )(tl2d, gathered, gathered, attMap_e, adjust2)


def kernel(Final_output, attMap_e, adjust2, traj, traj_len):
    del Final_output
    tlen = traj_len.astype(jnp.int32)
    traj2t = jnp.pad(traj[:, :, 1].astype(jnp.int32).T, ((0, HP - HIST), (0, 0)))
    idx_flat = _compute_idx(traj2t, tlen.reshape(1, B))
    gathered = _sc_gather(attMap_e, idx_flat)
    return _tc_matmul(gathered, attMap_e, adjust2, tlen.reshape(B, 1))


# speedup vs baseline: 1.0082x; 1.0082x over previous
import functools

import jax
import jax.numpy as jnp
from jax import lax
from jax.experimental import pallas as pl
from jax.experimental.pallas import tpu as pltpu
from jax.experimental.pallas import tpu_sc as plsc

B = 64
L = 8192
HIST = 50
FUSE_WEIGHT = 0.5

ROWS = 2 * B
RPW = 8
NWORK = ROWS // RPW

NK = 16
KB = L // NK


GROUPS = B // RPW


def _sc_gather(attMap_e, traj2f, tlen):
    mesh = plsc.VectorSubcoreMesh(core_axis_name="c", subcore_axis_name="s")

    @functools.partial(
        pl.kernel,
        out_type=jax.ShapeDtypeStruct((ROWS, L), jnp.float32),
        mesh=mesh,
        scratch_types=[
            pltpu.VMEM((16,), jnp.int32),
            pltpu.VMEM((16,), jnp.int32),
            pltpu.VMEM((16,), jnp.int32),
            pltpu.VMEM((16,), jnp.int32),
            pltpu.VMEM((RPW, L), jnp.float32),
            pltpu.SemaphoreType.DMA,
        ],
    )
    def gather_kernel(att_hbm, traj2f_hbm, tlen_hbm, out_hbm,
                      tl_v, fpos_v, ids_v, idx_v, rows_v, sem):
        wid = lax.axis_index("s") * 2 + lax.axis_index("c")

        @pl.when(wid < NWORK)
        def _():
            g = wid % GROUPS
            c = g // 2
            h = (g % 2) * RPW
            pltpu.sync_copy(tlen_hbm.at[pl.ds(c * 16, 16)], tl_v)
            tl = tl_v[...]
            lanes = lax.iota(jnp.int32, 16)
            bvec = c * 16 + lanes
            wrap = jnp.maximum(2 - tl, 0)
            kind = wid // GROUPS
            pos = tl - 1 - kind + HIST * kind * wrap
            fpos_v[...] = bvec * HIST + pos
            pltpu.async_copy(traj2f_hbm.at[fpos_v], ids_v, sem).wait()
            idx_v[...] = ids_v[...] - 1
            pltpu.async_copy(att_hbm.at[idx_v.at[pl.ds(h, RPW)]], rows_v,
                             sem).wait()
            pltpu.sync_copy(rows_v, out_hbm.at[pl.ds(wid * RPW, RPW)])

    return gather_kernel(attMap_e, traj2f, tlen)


def _tc_body(tl_ref, prob1_ref, probk_ref, att_ref, adj_ref, out_ref):
    k = pl.program_id(0)

    @pl.when(k == 0)
    def _():
        out_ref[...] = jnp.zeros_like(out_ref)

    a = probk_ref[...].astype(jnp.bfloat16)
    bm = att_ref[...].astype(jnp.bfloat16)
    out_ref[...] += jnp.dot(a, bm, preferred_element_type=jnp.float32)

    @pl.when(k == NK - 1)
    def _():
        acc = out_ref[...]
        mn2 = jnp.min(acc, axis=-1, keepdims=True)
        mx2 = jnp.max(acc, axis=-1, keepdims=True)
        p2 = (acc - mn2) / (mx2 - mn2)
        p1 = prob1_ref[...]
        mn1 = jnp.min(p1, axis=-1, keepdims=True)
        mx1 = jnp.max(p1, axis=-1, keepdims=True)
        y1 = (p1 - mn1) / (mx1 - mn1)
        adj = adj_ref[...]
        wn = (adj - jnp.min(adj)) / (jnp.max(adj) - jnp.min(adj))
        mask = (tl_ref[...] >= 2).astype(jnp.float32)
        out_ref[...] = y1 + mask * (FUSE_WEIGHT * wn) * p2


def _tc_matmul(gathered, attMap_e, adjust2, tl2d):
    return pl.pallas_call(
        _tc_body,
        grid=(NK,),
        in_specs=[
            pl.BlockSpec((B, 1), lambda k: (0, 0)),
            pl.BlockSpec((B, L), lambda k: (0, 0)),
            pl.BlockSpec((B, KB), lambda k: (1, k)),
            pl.BlockSpec((KB, L), lambda k: (k, 0)),
            pl.BlockSpec((1, L), lambda k: (0, 0)),
        ],
        out_specs=pl.BlockSpec((B, L), lambda k: (0, 0)),
        out_shape=jax.ShapeDtypeStruct((B, L), jnp.float32),
    )(tl2d, gathered, gathered, attMap_e, adjust2)


def kernel(Final_output, attMap_e, adjust2, traj, traj_len):
    del Final_output
    tlen = traj_len.astype(jnp.int32)
    traj2f = traj[:, :, 1].astype(jnp.int32).reshape(B * HIST)
    gathered = _sc_gather(attMap_e, traj2f, tlen)
    return _tc_matmul(gathered, attMap_e, adjust2, tlen.reshape(B, 1))


# speedup vs baseline: 1.0455x; 1.0370x over previous
import functools

import jax
import jax.numpy as jnp
from jax import lax
from jax.experimental import pallas as pl
from jax.experimental.pallas import tpu as pltpu
from jax.experimental.pallas import tpu_sc as plsc

B = 64
L = 8192
HIST = 50
FUSE_WEIGHT = 0.5

ROWS = 2 * B
NCHUNK = ROWS // 16

NK = 16
KB = L // NK


def _sc_indices(trajf, tlen):
    mesh = plsc.VectorSubcoreMesh(core_axis_name="c", subcore_axis_name="s")

    @functools.partial(
        pl.kernel,
        out_type=jax.ShapeDtypeStruct((ROWS,), jnp.int32),
        mesh=mesh,
        scratch_types=[
            pltpu.VMEM((16,), jnp.int32),
            pltpu.VMEM((16,), jnp.int32),
            pltpu.VMEM((16,), jnp.int32),
            pltpu.VMEM((16,), jnp.int32),
            pltpu.SemaphoreType.DMA,
        ],
    )
    def idx_kernel(trajf_hbm, tlen_hbm, out_hbm, tl_v, fpos_v, ids_v, idx_v,
                   sem):
        wid = lax.axis_index("s") * 2 + lax.axis_index("c")

        @pl.when(wid < NCHUNK)
        def _():
            kind = wid // 4
            c = wid % 4
            pltpu.sync_copy(tlen_hbm.at[pl.ds(c * 16, 16)], tl_v)
            tl = tl_v[...]
            bvec = c * 16 + lax.iota(jnp.int32, 16)
            wrap = jnp.maximum(2 - tl, 0)
            pos = tl - 1 - kind + HIST * kind * wrap
            fpos_v[...] = (bvec * HIST + pos) * 2 + 1
            pltpu.async_copy(trajf_hbm.at[fpos_v], ids_v, sem).wait()
            idx_v[...] = ids_v[...] - 1
            pltpu.sync_copy(idx_v, out_hbm.at[pl.ds(wid * 16, 16)])

    return idx_kernel(trajf, tlen)


def _tc_body(idx_ref, tl_ref, att_ref, att_any, adj_ref, out_ref,
             prob_v, prob1_v, sem_p, sem_p1):
    k = pl.program_id(0)

    @pl.when(k == 0)
    def _():
        def start(j, carry):
            i1 = idx_ref[j]
            i2 = idx_ref[B + j]
            pltpu.make_async_copy(att_any.at[pl.ds(i1, 1), :],
                                  prob1_v.at[pl.ds(j, 1), :], sem_p1).start()
            pltpu.make_async_copy(att_any.at[pl.ds(i2, 1), :],
                                  prob_v.at[pl.ds(j, 1), :], sem_p).start()
            return carry

        lax.fori_loop(0, B, start, 0)

        def drain(j, carry):
            pltpu.make_async_copy(att_any.at[pl.ds(0, 1), :],
                                  prob_v.at[pl.ds(j, 1), :], sem_p).wait()
            return carry

        lax.fori_loop(0, B, drain, 0)
        out_ref[...] = jnp.zeros_like(out_ref)

    a = prob_v[:, pl.ds(k * KB, KB)].astype(jnp.bfloat16)
    bm = att_ref[...].astype(jnp.bfloat16)
    out_ref[...] += jnp.dot(a, bm, preferred_element_type=jnp.float32)

    @pl.when(k == NK - 1)
    def _():
        def drain1(j, carry):
            pltpu.make_async_copy(att_any.at[pl.ds(0, 1), :],
                                  prob1_v.at[pl.ds(j, 1), :], sem_p1).wait()
            return carry

        lax.fori_loop(0, B, drain1, 0)
        acc = out_ref[...]
        mn2 = jnp.min(acc, axis=-1, keepdims=True)
        mx2 = jnp.max(acc, axis=-1, keepdims=True)
        p2 = (acc - mn2) / (mx2 - mn2)
        p1 = prob1_v[...]
        mn1 = jnp.min(p1, axis=-1, keepdims=True)
        mx1 = jnp.max(p1, axis=-1, keepdims=True)
        y1 = (p1 - mn1) / (mx1 - mn1)
        adj = adj_ref[...]
        wn = (adj - jnp.min(adj)) / (jnp.max(adj) - jnp.min(adj))
        mask = (tl_ref[...] >= 2).astype(jnp.float32)
        out_ref[...] = y1 + mask * (FUSE_WEIGHT * wn) * p2


def _tc_matmul(idx, attMap_e, adjust2, tl2d):
    return pl.pallas_call(
        _tc_body,
        grid=(NK,),
        in_specs=[
            pl.BlockSpec(memory_space=pltpu.SMEM),
            pl.BlockSpec((B, 1), lambda k: (0, 0)),
            pl.BlockSpec((KB, L), lambda k: (k, 0)),
            pl.BlockSpec(memory_space=pl.ANY),
            pl.BlockSpec((1, L), lambda k: (0, 0)),
        ],
        out_specs=pl.BlockSpec((B, L), lambda k: (0, 0)),
        out_shape=jax.ShapeDtypeStruct((B, L), jnp.float32),
        scratch_shapes=[
            pltpu.VMEM((B, L), jnp.float32),
            pltpu.VMEM((B, L), jnp.float32),
            pltpu.SemaphoreType.DMA,
            pltpu.SemaphoreType.DMA,
        ],
        compiler_params=pltpu.CompilerParams(
            vmem_limit_bytes=100 * 1024 * 1024,
        ),
    )(idx, tl2d, attMap_e, attMap_e, adjust2)


def kernel(Final_output, attMap_e, adjust2, traj, traj_len):
    del Final_output
    tlen = traj_len.astype(jnp.int32)
    trajf = traj.astype(jnp.int32).reshape(B * HIST * 2)
    idx = _sc_indices(trajf, tlen)
    return _tc_matmul(idx, attMap_e, adjust2, tlen.reshape(B, 1))


# speedup vs baseline: 1.2231x; 1.1699x over previous
import functools

import jax
import jax.numpy as jnp
from jax import lax
from jax.experimental import pallas as pl
from jax.experimental.pallas import tpu as pltpu
from jax.experimental.pallas import tpu_sc as plsc

B = 64
L = 8192
HIST = 50
FUSE_WEIGHT = 0.5

ROWS = 2 * B
NCHUNK = ROWS // 16

NK = 16
KB = L // NK


def _sc_indices(trajf, tlen):
    mesh = plsc.VectorSubcoreMesh(core_axis_name="c", subcore_axis_name="s")

    @functools.partial(
        pl.kernel,
        out_type=jax.ShapeDtypeStruct((ROWS,), jnp.int32),
        mesh=mesh,
        scratch_types=[
            pltpu.VMEM((16,), jnp.int32),
            pltpu.VMEM((16,), jnp.int32),
            pltpu.VMEM((16,), jnp.int32),
            pltpu.VMEM((16,), jnp.int32),
            pltpu.SemaphoreType.DMA,
        ],
    )
    def idx_kernel(trajf_hbm, tlen_hbm, out_hbm, tl_v, fpos_v, ids_v, idx_v,
                   sem):
        wid = lax.axis_index("s") * 2 + lax.axis_index("c")

        @pl.when(wid < NCHUNK)
        def _():
            kind = wid // 4
            c = wid % 4
            pltpu.sync_copy(tlen_hbm.at[pl.ds(c * 16, 16)], tl_v)
            tl = tl_v[...]
            bvec = c * 16 + lax.iota(jnp.int32, 16)
            wrap = jnp.maximum(2 - tl, 0)
            pos = tl - 1 - kind + HIST * kind * wrap
            fpos_v[...] = (bvec * HIST + pos) * 2 + 1
            pltpu.async_copy(trajf_hbm.at[fpos_v], ids_v, sem).wait()
            idx_v[...] = ids_v[...] - 1
            pltpu.sync_copy(idx_v, out_hbm.at[pl.ds(wid * 16, 16)])

    return idx_kernel(trajf, tlen)


def _tc_body(idx_ref, tl_ref, att_ref, att_any, adj_ref, out_ref,
             prob_v, prob1_v, sem_p, sem_p1):
    k = pl.program_id(0)

    @pl.when(k == 0)
    def _():
        def start(j, carry):
            tl = idx_ref[j]
            i1 = idx_ref[B + (j * HIST + tl - 1) * 2 + 1] - 1
            pos2 = tl - 2 + HIST * jnp.maximum(2 - tl, 0)
            i2 = idx_ref[B + (j * HIST + pos2) * 2 + 1] - 1
            pltpu.make_async_copy(att_any.at[pl.ds(i1, 1), :],
                                  prob1_v.at[pl.ds(j, 1), :], sem_p1).start()
            pltpu.make_async_copy(att_any.at[pl.ds(i2, 1), :],
                                  prob_v.at[pl.ds(j, 1), :], sem_p).start()
            return carry

        lax.fori_loop(0, B, start, 0)

        def drain(j, carry):
            pltpu.make_async_copy(att_any.at[pl.ds(0, 1), :],
                                  prob_v.at[pl.ds(j, 1), :], sem_p).wait()
            return carry

        lax.fori_loop(0, B, drain, 0)
        out_ref[...] = jnp.zeros_like(out_ref)

    a = prob_v[:, pl.ds(k * KB, KB)].astype(jnp.bfloat16)
    bm = att_ref[...].astype(jnp.bfloat16)
    out_ref[...] += jnp.dot(a, bm, preferred_element_type=jnp.float32)

    @pl.when(k == NK - 1)
    def _():
        def drain1(j, carry):
            pltpu.make_async_copy(att_any.at[pl.ds(0, 1), :],
                                  prob1_v.at[pl.ds(j, 1), :], sem_p1).wait()
            return carry

        lax.fori_loop(0, B, drain1, 0)
        acc = out_ref[...]
        mn2 = jnp.min(acc, axis=-1, keepdims=True)
        mx2 = jnp.max(acc, axis=-1, keepdims=True)
        p2 = (acc - mn2) / (mx2 - mn2)
        p1 = prob1_v[...]
        mn1 = jnp.min(p1, axis=-1, keepdims=True)
        mx1 = jnp.max(p1, axis=-1, keepdims=True)
        y1 = (p1 - mn1) / (mx1 - mn1)
        adj = adj_ref[...]
        wn = (adj - jnp.min(adj)) / (jnp.max(adj) - jnp.min(adj))
        mask = (tl_ref[...] >= 2).astype(jnp.float32)
        out_ref[...] = y1 + mask * (FUSE_WEIGHT * wn) * p2


def _tc_matmul(idx, attMap_e, adjust2, tl2d):
    return pl.pallas_call(
        _tc_body,
        grid=(NK,),
        in_specs=[
            pl.BlockSpec(memory_space=pltpu.SMEM),
            pl.BlockSpec((B, 1), lambda k: (0, 0)),
            pl.BlockSpec((KB, L), lambda k: (k, 0)),
            pl.BlockSpec(memory_space=pl.ANY),
            pl.BlockSpec((1, L), lambda k: (0, 0)),
        ],
        out_specs=pl.BlockSpec((B, L), lambda k: (0, 0)),
        out_shape=jax.ShapeDtypeStruct((B, L), jnp.float32),
        scratch_shapes=[
            pltpu.VMEM((B, L), jnp.float32),
            pltpu.VMEM((B, L), jnp.float32),
            pltpu.SemaphoreType.DMA,
            pltpu.SemaphoreType.DMA,
        ],
        compiler_params=pltpu.CompilerParams(
            vmem_limit_bytes=100 * 1024 * 1024,
        ),
    )(idx, tl2d, attMap_e, attMap_e, adjust2)


def kernel(Final_output, attMap_e, adjust2, traj, traj_len):
    del Final_output
    tlen = traj_len.astype(jnp.int32)
    trajf = traj.astype(jnp.int32).reshape(B * HIST * 2)
    meta = jnp.concatenate([tlen, trajf])
    return _tc_matmul(meta, attMap_e, adjust2, tlen.reshape(B, 1))
